# baseline (speedup 1.0000x reference)
import functools

import jax
import jax.numpy as jnp
from jax import lax
from jax.experimental import pallas as pl
from jax.experimental.pallas import tpu as pltpu
from jax.experimental.pallas import tpu_sc as plsc

BATCH_SIZE = 16384
D0 = 64

_info = plsc.get_sparse_core_info()
_NC, _NS = _info.num_cores, _info.num_subcores
_NW = _NC * _NS
_BPW = BATCH_SIZE // _NW
_CHUNK = 128
_NCHUNK = _BPW // _CHUNK


def _body(d0_hbm, d1_hbm, idx_hbm, out0_hbm, out1_hbm,
          idx_v, rows0_v, rows1_v, sem):
    wid = lax.axis_index("s") * _NC + lax.axis_index("c")
    base = wid * _BPW
    pltpu.sync_copy(idx_hbm.at[wid], idx_v)
    copies = []
    for j in range(_NCHUNK):
        sl = pl.ds(j * _CHUNK, _CHUNK)
        copies.append(pltpu.async_copy(d0_hbm.at[idx_v.at[j]], rows0_v.at[sl], sem))
        copies.append(pltpu.async_copy(d1_hbm.at[idx_v.at[j]], rows1_v.at[sl], sem))
    for c in copies:
        c.wait()
    pltpu.sync_copy(rows0_v, out0_hbm.at[pl.ds(base, _BPW)])
    pltpu.sync_copy(rows1_v, out1_hbm.at[pl.ds(base, _BPW)])


@jax.jit
def _run(data0, data1, idx):
    mesh = plsc.VectorSubcoreMesh(core_axis_name="c", subcore_axis_name="s")
    f = functools.partial(
        pl.kernel,
        mesh=mesh,
        out_type=(
            jax.ShapeDtypeStruct((BATCH_SIZE, D0), jnp.float32),
            jax.ShapeDtypeStruct((BATCH_SIZE,), jnp.float32),
        ),
        scratch_types=[
            pltpu.VMEM((_NCHUNK, _CHUNK), jnp.int32),
            pltpu.VMEM((_BPW, D0), jnp.float32),
            pltpu.VMEM((_BPW,), jnp.float32),
            pltpu.SemaphoreType.DMA,
        ],
        compiler_params=pltpu.CompilerParams(use_tc_tiling_on_sc=False),
    )(_body)
    out0, out1 = f(data0, data1.reshape(-1), idx)
    return out0, out1.reshape(BATCH_SIZE, 1)


def kernel(data0, data1, step):
    loader_key = jax.random.key(42)
    key = jax.random.fold_in(loader_key, step)
    idx = jax.random.randint(key, (BATCH_SIZE,), minval=0,
                             maxval=data0.shape[0], dtype=jnp.int32)
    return _run(data0, data1, idx.reshape(_NW, _NCHUNK, _CHUNK))
